# Initial kernel scaffold; baseline (speedup 1.0000x reference)
import jax
import jax.numpy as jnp
from jax.experimental import pallas as pl


def kernel(x, edge_attr, Wi, bi, We, be, Wg1, bg1, Wg2, bg2, Wqkv, bqkv, Wo, bo, Wm1, bm1, Wm2, bm2, Wh1, bh1, Wh2, bh2, edge_index, batch):
    raise NotImplementedError("write your pallas kernel here")


# speedup vs baseline: 18.2549x; 18.2549x over previous
import functools
import math

import jax
import jax.numpy as jnp
from jax import lax
from jax.experimental import pallas as pl

_N_PER = 32
_E_PER = 64
_WL = 20
_NL = 3
_HEADS = 4
_G = 8
_CH = 4
_NC = _G // _CH
_RSBN = 1.0 / math.sqrt(1.0 + 1e-5)


def _fused_kernel(x_ref, ea_ref, src_ref, dst_ref, mask_ref, eye_ref,
                  Wi_ref, bi_ref, We_ref, be_ref,
                  Wg1_ref, bg1_ref, Wg2_ref, bg2_ref,
                  Wqkv_ref, bqkv_ref, Wo_ref, bo_ref,
                  Wm1_ref, bm1_ref, Wm2_ref, bm2_ref,
                  Wh1_ref, bh1_ref, Wh2t_ref, bh2_ref, out_ref):
    f32 = jnp.float32
    npc = _CH * _N_PER
    epc = _CH * _E_PER
    hid = Wi_ref.shape[1]
    dh = hid // _HEADS

    x = x_ref[...]
    ea = ea_ref[...]
    src = src_ref[...]
    dst = dst_ref[...]
    mask = mask_ref[...]
    eye = eye_ref[...]

    src_oh = []
    dst_oh = []
    pe_chunks = []
    for c in range(_NC):
        s = src[c * _CH:(c + 1) * _CH]
        d = dst[c * _CH:(c + 1) * _CH]
        n_iota = lax.broadcasted_iota(jnp.int32, (_CH, _E_PER, npc), 2)
        m_iota = lax.broadcasted_iota(jnp.int32, (_CH, _E_PER, npc), 0)
        soh = (s[:, :, None] + _N_PER * m_iota == n_iota)
        doh = (d[:, :, None] + _N_PER * m_iota == n_iota)
        soh = soh.astype(f32).reshape(epc, npc)
        doh = doh.astype(f32).reshape(epc, npc)
        src_oh.append(soh)
        dst_oh.append(doh)
        deg = jnp.sum(soh, axis=0, keepdims=True) + 1.0
        inv = 1.0 / deg
        contrib = soh * inv
        M = jnp.dot(doh.T, contrib, preferred_element_type=f32)
        M = M + eye * inv
        p = eye
        cols = []
        for _ in range(_WL):
            p = jnp.dot(M, p, preferred_element_type=f32)
            cols.append(jnp.sum(p * eye, axis=1, keepdims=True))
        pe_chunks.append(jnp.concatenate(cols, axis=1))
    pe = jnp.concatenate(pe_chunks, axis=0)

    h = jnp.dot(jnp.concatenate([x, pe], axis=1), Wi_ref[...],
                preferred_element_type=f32) + bi_ref[...]
    ee = jnp.dot(ea, We_ref[...], preferred_element_type=f32) + be_ref[...]

    scale = 1.0 / math.sqrt(dh)
    for l in range(_NL):
        aggs = []
        for c in range(_NC):
            hc = h[c * npc:(c + 1) * npc]
            eec = ee[c * epc:(c + 1) * epc]
            msg = jnp.maximum(
                jnp.dot(src_oh[c], hc, preferred_element_type=f32) + eec, 0.0)
            aggs.append(jnp.dot(dst_oh[c].T, msg, preferred_element_type=f32))
        hl = h + jnp.concatenate(aggs, axis=0)
        hl = jnp.dot(
            jnp.maximum(jnp.dot(hl, Wg1_ref[l], preferred_element_type=f32)
                        + bg1_ref[l][None, :], 0.0),
            Wg2_ref[l], preferred_element_type=f32) + bg2_ref[l][None, :]
        hl = (hl + h) * _RSBN

        qkv = jnp.dot(h, Wqkv_ref[l], preferred_element_type=f32) \
            + bqkv_ref[l][None, :]
        att_rows = []
        for c in range(_NC):
            rows = qkv[c * npc:(c + 1) * npc]
            heads = []
            for t in range(_HEADS):
                q = rows[:, t * dh:(t + 1) * dh]
                k = rows[:, hid + t * dh:hid + (t + 1) * dh]
                v = rows[:, 2 * hid + t * dh:2 * hid + (t + 1) * dh]
                sc = jnp.dot(q, k.T, preferred_element_type=f32) * scale + mask
                sc = sc - jnp.max(sc, axis=1, keepdims=True)
                e = jnp.exp(sc)
                a = e / jnp.sum(e, axis=1, keepdims=True)
                heads.append(jnp.dot(a, v, preferred_element_type=f32))
            att_rows.append(jnp.concatenate(heads, axis=1))
        att = jnp.concatenate(att_rows, axis=0)
        ha = jnp.dot(att, Wo_ref[l], preferred_element_type=f32) \
            + bo_ref[l][None, :]
        ha = (ha + h) * _RSBN

        out = hl + ha
        mlp = jnp.dot(
            jnp.maximum(jnp.dot(out, Wm1_ref[l], preferred_element_type=f32)
                        + bm1_ref[l][None, :], 0.0),
            Wm2_ref[l], preferred_element_type=f32) + bm2_ref[l][None, :]
        h = (out + mlp) * _RSBN

    pooled = jnp.mean(h.reshape(_G, _N_PER, hid), axis=1)
    r = jnp.maximum(jnp.dot(pooled, Wh1_ref[...], preferred_element_type=f32)
                    + bh1_ref[...], 0.0)
    y = jnp.sum(r * Wh2t_ref[...], axis=1, keepdims=True)
    out_ref[...] = jnp.broadcast_to(y, out_ref.shape) + bh2_ref[...]


def kernel(x, edge_attr, Wi, bi, We, be, Wg1, bg1, Wg2, bg2, Wqkv, bqkv,
           Wo, bo, Wm1, bm1, Wm2, bm2, Wh1, bh1, Wh2, bh2, edge_index, batch):
    n, d_in = x.shape
    hid = Wi.shape[1]
    n_mol = n // _N_PER
    grid = n_mol // _G
    npc = _CH * _N_PER

    src2 = (edge_index[0] % _N_PER).astype(jnp.int32).reshape(n_mol, _E_PER)
    dst2 = (edge_index[1] % _N_PER).astype(jnp.int32).reshape(n_mol, _E_PER)

    mol_id = jnp.arange(npc, dtype=jnp.int32) // _N_PER
    mask = jnp.where(mol_id[:, None] == mol_id[None, :], 0.0, -1e30)
    mask = mask.astype(jnp.float32)
    eye = jnp.eye(npc, dtype=jnp.float32)

    const2 = lambda s: pl.BlockSpec(s, lambda i: (0, 0))
    const3 = lambda s: pl.BlockSpec(s, lambda i: (0, 0, 0))

    out = pl.pallas_call(
        _fused_kernel,
        grid=(grid,),
        in_specs=[
            pl.BlockSpec((_G * _N_PER, d_in), lambda i: (i, 0)),
            pl.BlockSpec((_G * _E_PER, edge_attr.shape[1]), lambda i: (i, 0)),
            pl.BlockSpec((_G, _E_PER), lambda i: (i, 0)),
            pl.BlockSpec((_G, _E_PER), lambda i: (i, 0)),
            const2(mask.shape), const2(eye.shape),
            const2(Wi.shape), const2((1, hid)),
            const2(We.shape), const2((1, hid)),
            const3(Wg1.shape), const2(bg1.shape),
            const3(Wg2.shape), const2(bg2.shape),
            const3(Wqkv.shape), const2(bqkv.shape),
            const3(Wo.shape), const2(bo.shape),
            const3(Wm1.shape), const2(bm1.shape),
            const3(Wm2.shape), const2(bm2.shape),
            const2(Wh1.shape), const2((1, Wh1.shape[1])),
            const2((1, Wh2.shape[0])),
            const2((1, hid)),
        ],
        out_specs=pl.BlockSpec((_G, hid), lambda i: (i, 0)),
        out_shape=jax.ShapeDtypeStruct((n_mol, hid), jnp.float32),
    )(
        x, edge_attr, src2, dst2, mask, eye,
        Wi, bi.reshape(1, -1), We, be.reshape(1, -1),
        Wg1, bg1, Wg2, bg2, Wqkv, bqkv, Wo, bo,
        Wm1, bm1, Wm2, bm2,
        Wh1, bh1.reshape(1, -1), Wh2.reshape(1, -1),
        jnp.broadcast_to(bh2.reshape(1, 1), (1, hid)),
    )
    return out[:, 0]


# speedup vs baseline: 33.3565x; 1.8273x over previous
import math

import jax
import jax.numpy as jnp
from jax import lax
from jax.experimental import pallas as pl

_N_PER = 32
_E_PER = 64
_WL = 20
_NL = 3
_HEADS = 4
_G = 8
_NPG = _G * _N_PER
_EPG = _G * _E_PER
_RSBN = 1.0 / math.sqrt(1.0 + 1e-5)


def _fused_kernel(x_ref, ea_ref, src_ref, dst_ref, mask_ref, eye_ref,
                  Wi_ref, bi_ref, We_ref, be_ref,
                  Wg1_ref, bg1_ref, Wg2_ref, bg2_ref,
                  Wqkv_ref, bqkv_ref, Wo_ref, bo_ref,
                  Wm1_ref, bm1_ref, Wm2_ref, bm2_ref,
                  Wh1_ref, bh1_ref, Wh2t_ref, bh2_ref, out_ref):
    f32 = jnp.float32
    bf16 = jnp.bfloat16
    hid = Wi_ref.shape[1]
    dh = hid // _HEADS

    x = x_ref[...]
    ea = ea_ref[...]
    src = src_ref[...]
    dst = dst_ref[...]
    mask = mask_ref[...]
    eye = eye_ref[...]

    n_iota = lax.broadcasted_iota(jnp.int32, (_G, _E_PER, _NPG), 2)
    m_iota = lax.broadcasted_iota(jnp.int32, (_G, _E_PER, _NPG), 0)
    soh = (src[:, :, None] + _N_PER * m_iota == n_iota)
    doh = (dst[:, :, None] + _N_PER * m_iota == n_iota)
    soh = soh.astype(f32).reshape(_EPG, _NPG)
    doh = doh.astype(f32).reshape(_EPG, _NPG)
    soh_b = soh.astype(bf16)
    doh_b = doh.astype(bf16)

    deg = jnp.sum(soh, axis=0, keepdims=True) + 1.0
    inv = 1.0 / deg
    m_raw = jnp.dot(doh_b.T, soh_b, preferred_element_type=f32)
    M = (m_raw + eye) * inv

    def diag_row(p):
        return jnp.sum(p * eye, axis=0, keepdims=True)

    Mb = M.astype(bf16)
    M2 = jnp.dot(Mb, Mb, preferred_element_type=f32)
    M2b = M2.astype(bf16)
    M3 = jnp.dot(M2b, Mb, preferred_element_type=f32)
    M4 = jnp.dot(M2b, M2b, preferred_element_type=f32)
    cols = [None] * _WL
    cols[0], cols[1], cols[2], cols[3] = (
        diag_row(M), diag_row(M2), diag_row(M3), diag_row(M4))
    M4b = M4.astype(bf16)
    seeds = [Mb, M2b, M3.astype(bf16), M4b]
    for r in range(4):
        q = seeds[r]
        for s in range(1, 5):
            t = r + 4 * s
            if t >= _WL:
                break
            qf = jnp.dot(q, M4b, preferred_element_type=f32)
            cols[t] = diag_row(qf)
            q = qf.astype(bf16)
    pe_t = jnp.concatenate(cols + [jnp.zeros((12, _NPG), f32)], axis=0)
    pe = pe_t.T[:, :_WL]

    h = jnp.dot(jnp.concatenate([x, pe], axis=1), Wi_ref[...],
                preferred_element_type=f32) + bi_ref[...]
    ee = jnp.dot(ea, We_ref[...], preferred_element_type=f32) + be_ref[...]

    scale = 1.0 / math.sqrt(dh)
    for l in range(_NL):
        msg = jnp.maximum(
            jnp.dot(soh, h, preferred_element_type=f32) + ee, 0.0)
        agg = jnp.dot(doh.T, msg, preferred_element_type=f32)
        hl = h + agg
        hl = jnp.dot(
            jnp.maximum(jnp.dot(hl, Wg1_ref[l], preferred_element_type=f32)
                        + bg1_ref[l][None, :], 0.0),
            Wg2_ref[l], preferred_element_type=f32) + bg2_ref[l][None, :]
        hl = (hl + h) * _RSBN

        qkv = jnp.dot(h, Wqkv_ref[l], preferred_element_type=f32) \
            + bqkv_ref[l][None, :]
        heads = []
        for t in range(_HEADS):
            q = qkv[:, t * dh:(t + 1) * dh]
            k = qkv[:, hid + t * dh:hid + (t + 1) * dh]
            v = qkv[:, 2 * hid + t * dh:2 * hid + (t + 1) * dh]
            sc = jnp.dot(q, k.T, preferred_element_type=f32) * scale + mask
            sc = sc - jnp.max(sc, axis=1, keepdims=True)
            e = jnp.exp(sc)
            a = e / jnp.sum(e, axis=1, keepdims=True)
            heads.append(jnp.dot(a, v, preferred_element_type=f32))
        att = jnp.concatenate(heads, axis=1)
        ha = jnp.dot(att, Wo_ref[l], preferred_element_type=f32) \
            + bo_ref[l][None, :]
        ha = (ha + h) * _RSBN

        out = hl + ha
        mlp = jnp.dot(
            jnp.maximum(jnp.dot(out, Wm1_ref[l], preferred_element_type=f32)
                        + bm1_ref[l][None, :], 0.0),
            Wm2_ref[l], preferred_element_type=f32) + bm2_ref[l][None, :]
        h = (out + mlp) * _RSBN

    pooled = jnp.mean(h.reshape(_G, _N_PER, hid), axis=1)
    r = jnp.maximum(jnp.dot(pooled, Wh1_ref[...], preferred_element_type=f32)
                    + bh1_ref[...], 0.0)
    y = jnp.sum(r * Wh2t_ref[...], axis=1, keepdims=True)
    out_ref[...] = jnp.broadcast_to(y, out_ref.shape) + bh2_ref[...]


def kernel(x, edge_attr, Wi, bi, We, be, Wg1, bg1, Wg2, bg2, Wqkv, bqkv,
           Wo, bo, Wm1, bm1, Wm2, bm2, Wh1, bh1, Wh2, bh2, edge_index, batch):
    n, d_in = x.shape
    hid = Wi.shape[1]
    n_mol = n // _N_PER
    grid = n_mol // _G

    src2 = (edge_index[0] % _N_PER).astype(jnp.int32).reshape(n_mol, _E_PER)
    dst2 = (edge_index[1] % _N_PER).astype(jnp.int32).reshape(n_mol, _E_PER)

    mol_id = jnp.arange(_NPG, dtype=jnp.int32) // _N_PER
    mask = jnp.where(mol_id[:, None] == mol_id[None, :], 0.0, -1e30)
    mask = mask.astype(jnp.float32)
    eye = jnp.eye(_NPG, dtype=jnp.float32)

    const2 = lambda s: pl.BlockSpec(s, lambda i: (0, 0))
    const3 = lambda s: pl.BlockSpec(s, lambda i: (0, 0, 0))

    out = pl.pallas_call(
        _fused_kernel,
        grid=(grid,),
        in_specs=[
            pl.BlockSpec((_NPG, d_in), lambda i: (i, 0)),
            pl.BlockSpec((_EPG, edge_attr.shape[1]), lambda i: (i, 0)),
            pl.BlockSpec((_G, _E_PER), lambda i: (i, 0)),
            pl.BlockSpec((_G, _E_PER), lambda i: (i, 0)),
            const2(mask.shape), const2(eye.shape),
            const2(Wi.shape), const2((1, hid)),
            const2(We.shape), const2((1, hid)),
            const3(Wg1.shape), const2(bg1.shape),
            const3(Wg2.shape), const2(bg2.shape),
            const3(Wqkv.shape), const2(bqkv.shape),
            const3(Wo.shape), const2(bo.shape),
            const3(Wm1.shape), const2(bm1.shape),
            const3(Wm2.shape), const2(bm2.shape),
            const2(Wh1.shape), const2((1, Wh1.shape[1])),
            const2((1, Wh2.shape[0])),
            const2((1, hid)),
        ],
        out_specs=pl.BlockSpec((_G, hid), lambda i: (i, 0)),
        out_shape=jax.ShapeDtypeStruct((n_mol, hid), jnp.float32),
    )(
        x, edge_attr, src2, dst2, mask, eye,
        Wi, bi.reshape(1, -1), We, be.reshape(1, -1),
        Wg1, bg1, Wg2, bg2, Wqkv, bqkv, Wo, bo,
        Wm1, bm1, Wm2, bm2,
        Wh1, bh1.reshape(1, -1), Wh2.reshape(1, -1),
        jnp.broadcast_to(bh2.reshape(1, 1), (1, hid)),
    )
    return out[:, 0]


# speedup vs baseline: 33.7513x; 1.0118x over previous
import math

import jax
import jax.numpy as jnp
from jax import lax
from jax.experimental import pallas as pl

_N_PER = 32
_E_PER = 64
_WL = 20
_NL = 3
_HEADS = 4
_G = 8
_NPG = _G * _N_PER
_EPG = _G * _E_PER
_RSBN = 1.0 / math.sqrt(1.0 + 1e-5)


def _fused_kernel(x_ref, ea_ref, src_ref, dst_ref, mask_ref, eye_ref,
                  Wi_ref, bi_ref, We_ref, be_ref,
                  Wg1_ref, bg1_ref, Wg2_ref, bg2_ref,
                  Wqkv_ref, bqkv_ref, Wo_ref, bo_ref,
                  Wm1_ref, bm1_ref, Wm2_ref, bm2_ref,
                  Wh1_ref, bh1_ref, Wh2t_ref, bh2_ref, out_ref):
    f32 = jnp.float32
    bf16 = jnp.bfloat16
    hid = Wi_ref.shape[1]
    dh = hid // _HEADS

    x = x_ref[...]
    ea = ea_ref[...]
    src = src_ref[...]
    dst = dst_ref[...]
    mask = mask_ref[...]
    eye = eye_ref[...]

    n_iota = lax.broadcasted_iota(jnp.int32, (_G, _E_PER, _NPG), 2)
    m_iota = lax.broadcasted_iota(jnp.int32, (_G, _E_PER, _NPG), 0)
    soh = (src[:, :, None] + _N_PER * m_iota == n_iota)
    doh = (dst[:, :, None] + _N_PER * m_iota == n_iota)
    soh = soh.astype(f32).reshape(_EPG, _NPG)
    doh = doh.astype(f32).reshape(_EPG, _NPG)
    soh_b = soh.astype(bf16)
    doh_b = doh.astype(bf16)

    deg = jnp.sum(soh, axis=0, keepdims=True) + 1.0
    inv = 1.0 / deg
    m_raw = jnp.dot(doh_b.T, soh_b, preferred_element_type=f32)
    M = (m_raw + eye) * inv

    def diag_row(p):
        return jnp.sum(p * eye, axis=0, keepdims=True)

    Mb = M.astype(bf16)
    M2 = jnp.dot(Mb, Mb, preferred_element_type=f32)
    M2b = M2.astype(bf16)
    M3 = jnp.dot(M2b, Mb, preferred_element_type=f32)
    M4 = jnp.dot(M2b, M2b, preferred_element_type=f32)
    cols = [None] * _WL
    cols[0], cols[1], cols[2], cols[3] = (
        diag_row(M), diag_row(M2), diag_row(M3), diag_row(M4))
    M4b = M4.astype(bf16)
    seeds = [Mb, M2b, M3.astype(bf16), M4b]
    for r in range(4):
        q = seeds[r]
        for s in range(1, 5):
            t = r + 4 * s
            if t >= _WL:
                break
            qf = jnp.dot(q, M4b, preferred_element_type=f32)
            cols[t] = diag_row(qf)
            q = qf.astype(bf16)
    pe_t = jnp.concatenate(cols + [jnp.zeros((12, _NPG), f32)], axis=0)
    pe = pe_t.T[:, :_WL]

    h = jnp.dot(jnp.concatenate([x, pe.astype(bf16)], axis=1), Wi_ref[...],
                preferred_element_type=f32) + bi_ref[...]
    ee = jnp.dot(ea, We_ref[...], preferred_element_type=f32) + be_ref[...]

    scale = 1.0 / math.sqrt(dh)
    for l in range(_NL):
        hb = h.astype(bf16)
        msg = jnp.maximum(
            jnp.dot(soh_b, hb, preferred_element_type=f32) + ee, 0.0)
        agg = jnp.dot(doh_b.T, msg.astype(bf16), preferred_element_type=f32)
        hl = h + agg
        hl = jnp.dot(
            jnp.maximum(jnp.dot(hl.astype(bf16), Wg1_ref[l],
                                preferred_element_type=f32)
                        + bg1_ref[l][None, :], 0.0).astype(bf16),
            Wg2_ref[l], preferred_element_type=f32) + bg2_ref[l][None, :]
        hl = (hl + h) * _RSBN

        qkv = jnp.dot(hb, Wqkv_ref[l], preferred_element_type=f32) \
            + bqkv_ref[l][None, :]
        heads = []
        for t in range(_HEADS):
            q = qkv[:, t * dh:(t + 1) * dh].astype(bf16)
            k = qkv[:, hid + t * dh:hid + (t + 1) * dh].astype(bf16)
            v = qkv[:, 2 * hid + t * dh:2 * hid + (t + 1) * dh].astype(bf16)
            sc = jnp.dot(q, k.T, preferred_element_type=f32) * scale + mask
            sc = sc - jnp.max(sc, axis=1, keepdims=True)
            e = jnp.exp(sc)
            a = (e / jnp.sum(e, axis=1, keepdims=True)).astype(bf16)
            heads.append(jnp.dot(a, v, preferred_element_type=f32))
        att = jnp.concatenate(heads, axis=1).astype(bf16)
        ha = jnp.dot(att, Wo_ref[l], preferred_element_type=f32) \
            + bo_ref[l][None, :]
        ha = (ha + h) * _RSBN

        out = hl + ha
        mlp = jnp.dot(
            jnp.maximum(jnp.dot(out.astype(bf16), Wm1_ref[l],
                                preferred_element_type=f32)
                        + bm1_ref[l][None, :], 0.0).astype(bf16),
            Wm2_ref[l], preferred_element_type=f32) + bm2_ref[l][None, :]
        h = (out + mlp) * _RSBN

    pooled = jnp.mean(h.reshape(_G, _N_PER, hid), axis=1)
    r = jnp.maximum(jnp.dot(pooled, Wh1_ref[...], preferred_element_type=f32)
                    + bh1_ref[...], 0.0)
    y = jnp.sum(r * Wh2t_ref[...], axis=1, keepdims=True)
    out_ref[...] = jnp.broadcast_to(y, out_ref.shape) + bh2_ref[...]


def kernel(x, edge_attr, Wi, bi, We, be, Wg1, bg1, Wg2, bg2, Wqkv, bqkv,
           Wo, bo, Wm1, bm1, Wm2, bm2, Wh1, bh1, Wh2, bh2, edge_index, batch):
    n, d_in = x.shape
    hid = Wi.shape[1]
    n_mol = n // _N_PER
    grid = n_mol // _G

    src2 = (edge_index[0] % _N_PER).astype(jnp.int32).reshape(n_mol, _E_PER)
    dst2 = (edge_index[1] % _N_PER).astype(jnp.int32).reshape(n_mol, _E_PER)

    mol_id = jnp.arange(_NPG, dtype=jnp.int32) // _N_PER
    mask = jnp.where(mol_id[:, None] == mol_id[None, :], 0.0, -1e30)
    mask = mask.astype(jnp.float32)
    eye = jnp.eye(_NPG, dtype=jnp.float32)

    const2 = lambda s: pl.BlockSpec(s, lambda i: (0, 0))
    const3 = lambda s: pl.BlockSpec(s, lambda i: (0, 0, 0))

    out = pl.pallas_call(
        _fused_kernel,
        grid=(grid,),
        in_specs=[
            pl.BlockSpec((_NPG, d_in), lambda i: (i, 0)),
            pl.BlockSpec((_EPG, edge_attr.shape[1]), lambda i: (i, 0)),
            pl.BlockSpec((_G, _E_PER), lambda i: (i, 0)),
            pl.BlockSpec((_G, _E_PER), lambda i: (i, 0)),
            const2(mask.shape), const2(eye.shape),
            const2(Wi.shape), const2((1, hid)),
            const2(We.shape), const2((1, hid)),
            const3(Wg1.shape), const2(bg1.shape),
            const3(Wg2.shape), const2(bg2.shape),
            const3(Wqkv.shape), const2(bqkv.shape),
            const3(Wo.shape), const2(bo.shape),
            const3(Wm1.shape), const2(bm1.shape),
            const3(Wm2.shape), const2(bm2.shape),
            const2(Wh1.shape), const2((1, Wh1.shape[1])),
            const2((1, Wh2.shape[0])),
            const2((1, hid)),
        ],
        out_specs=pl.BlockSpec((_G, hid), lambda i: (i, 0)),
        out_shape=jax.ShapeDtypeStruct((n_mol, hid), jnp.float32),
    )(
        x.astype(jnp.bfloat16), edge_attr.astype(jnp.bfloat16),
        src2, dst2, mask, eye,
        Wi.astype(jnp.bfloat16), bi.reshape(1, -1),
        We.astype(jnp.bfloat16), be.reshape(1, -1),
        Wg1.astype(jnp.bfloat16), bg1, Wg2.astype(jnp.bfloat16), bg2,
        Wqkv.astype(jnp.bfloat16), bqkv, Wo.astype(jnp.bfloat16), bo,
        Wm1.astype(jnp.bfloat16), bm1, Wm2.astype(jnp.bfloat16), bm2,
        Wh1, bh1.reshape(1, -1), Wh2.reshape(1, -1),
        jnp.broadcast_to(bh2.reshape(1, 1), (1, hid)),
    )
    return out[:, 0]
